# Initial kernel scaffold; baseline (speedup 1.0000x reference)
import jax
import jax.numpy as jnp
from jax.experimental import pallas as pl


def kernel(input, target, idx_expanded):
    raise NotImplementedError("write your pallas kernel here")


# speedup vs baseline: 1.3606x; 1.3606x over previous
import functools

import jax
import jax.numpy as jnp
from jax import lax
from jax.experimental import pallas as pl
from jax.experimental.pallas import tpu as pltpu
from jax.experimental.pallas import tpu_sc as plsc

BN = 32
Q = 64
P = 64
R = 64
LANES = 16
TOTAL = BN * Q * P * R


@functools.partial(
    pl.kernel,
    out_type=jax.ShapeDtypeStruct((BN, LANES), jnp.float32),
    mesh=plsc.VectorSubcoreMesh(core_axis_name="c", subcore_axis_name="s"),
    compiler_params=pltpu.CompilerParams(needs_layout_passes=False),
    scratch_types=[
        pltpu.VMEM((Q * P,), jnp.float32),
        pltpu.VMEM((P, R), jnp.int32),
        pltpu.VMEM((P, R), jnp.float32),
        pltpu.VMEM((LANES,), jnp.float32),
    ],
)
def _pjc_sc(x_hbm, tgt_hbm, idx_hbm, out_hbm, xq_v, idx_v, tgt_v, acc_v):
    w = lax.axis_index("s") * 2 + lax.axis_index("c")

    def q_body(u, acc):
        pltpu.sync_copy(x_hbm.at[w * Q + u], xq_v)
        pltpu.sync_copy(idx_hbm.at[w, :, u, :], idx_v)
        pltpu.sync_copy(tgt_hbm.at[w, :, u, :], tgt_v)

        def p_body(p, acc_in):
            a = acc_in
            for c in range(R // LANES):
                iv = idx_v[p, pl.ds(c * LANES, LANES)]
                fl = iv * P + p
                g = plsc.load_gather(xq_v, [fl])
                t = tgt_v[p, pl.ds(c * LANES, LANES)]
                d = g - t
                a = a + d * d
            return a

        return lax.fori_loop(0, P, p_body, acc)

    acc = lax.fori_loop(0, Q, q_body, jnp.zeros((LANES,), jnp.float32))
    acc_v[...] = acc * (1.0 / TOTAL)
    pltpu.sync_copy(acc_v, out_hbm.at[w])


def kernel(input, target, idx_expanded):
    x = input.reshape(BN * Q, P * R)
    tgt = target.reshape(BN, P, Q, R)
    idx = idx_expanded.reshape(BN, P, Q, R)
    partial = _pjc_sc(x, tgt, idx)
    return jnp.sum(partial)


# speedup vs baseline: 2.4217x; 1.7798x over previous
import functools

import jax
import jax.numpy as jnp
from jax import lax
from jax.experimental import pallas as pl
from jax.experimental.pallas import tpu as pltpu
from jax.experimental.pallas import tpu_sc as plsc

BN = 32
Q = 64
P = 64
R = 64
LANES = 16
NCHUNK = R // LANES
TOTAL = BN * Q * P * R


@functools.partial(
    pl.kernel,
    out_type=jax.ShapeDtypeStruct((BN, LANES), jnp.float32),
    mesh=plsc.VectorSubcoreMesh(core_axis_name="c", subcore_axis_name="s"),
    compiler_params=pltpu.CompilerParams(needs_layout_passes=False),
    scratch_types=[
        pltpu.VMEM((Q * P,), jnp.float32),
        pltpu.VMEM((Q * P,), jnp.float32),
        pltpu.VMEM((P, R), jnp.int32),
        pltpu.VMEM((P, R), jnp.int32),
        pltpu.VMEM((P, R), jnp.float32),
        pltpu.VMEM((P, R), jnp.float32),
        pltpu.VMEM((LANES,), jnp.float32),
        pltpu.SemaphoreType.DMA,
        pltpu.SemaphoreType.DMA,
    ],
)
def _pjc_sc(x_hbm, tgt_hbm, idx_hbm, out_hbm,
            xq0, xq1, idx0, idx1, tgt0, tgt1, acc_v, sem0, sem1):
    w = lax.axis_index("s") * 2 + lax.axis_index("c")

    def issue(u, xq, idxb, tgtb, sem):
        pltpu.async_copy(x_hbm.at[w * Q + u], xq, sem)
        pltpu.async_copy(idx_hbm.at[w, :, u, :], idxb, sem)
        pltpu.async_copy(tgt_hbm.at[w, :, u, :], tgtb, sem)

    def drain(u, xq, idxb, tgtb, sem):
        pltpu.make_async_copy(x_hbm.at[w * Q + u], xq, sem).wait()
        pltpu.make_async_copy(idx_hbm.at[w, :, u, :], idxb, sem).wait()
        pltpu.make_async_copy(tgt_hbm.at[w, :, u, :], tgtb, sem).wait()

    def unit_compute(xq, idxb, tgtb, accs):
        def body(p, a):
            a = list(a)
            for c in range(NCHUNK):
                iv = idxb[p, pl.ds(c * LANES, LANES)]
                fl = iv * R + p
                g = plsc.load_gather(xq, [fl])
                t = tgtb[p, pl.ds(c * LANES, LANES)]
                d = g - t
                a[c] = a[c] + d * d
            return tuple(a)
        return plsc.parallel_loop(0, P, unroll=4, carry=accs)(body)

    issue(0, xq0, idx0, tgt0, sem0)
    issue(1, xq1, idx1, tgt1, sem1)

    def pair_body(i, accs):
        u0 = 2 * i
        drain(u0, xq0, idx0, tgt0, sem0)
        accs = unit_compute(xq0, idx0, tgt0, accs)

        @pl.when(u0 + 2 < Q)
        def _():
            issue(u0 + 2, xq0, idx0, tgt0, sem0)

        u1 = u0 + 1
        drain(u1, xq1, idx1, tgt1, sem1)
        accs = unit_compute(xq1, idx1, tgt1, accs)

        @pl.when(u1 + 2 < Q)
        def _():
            issue(u1 + 2, xq1, idx1, tgt1, sem1)

        return accs

    zero = jnp.zeros((LANES,), jnp.float32)
    accs = lax.fori_loop(0, Q // 2, pair_body, (zero, zero, zero, zero))
    total = (accs[0] + accs[1]) + (accs[2] + accs[3])
    acc_v[...] = total * (1.0 / TOTAL)
    pltpu.sync_copy(acc_v, out_hbm.at[w])


def kernel(input, target, idx_expanded):
    x = input.reshape(BN * Q, P * R)
    tgt = target.reshape(BN, P, Q, R)
    idx = idx_expanded.reshape(BN, P, Q, R)
    partial = _pjc_sc(x, tgt, idx)
    return jnp.sum(partial)
